# Initial kernel scaffold; baseline (speedup 1.0000x reference)
import jax
import jax.numpy as jnp
from jax.experimental import pallas as pl


def kernel(x, edge_index, batch, mlp_W1, mlp_b1, mlp_g1, mlp_be1, mlp_W2, mlp_b2, bn_g, bn_b, cls_W1, cls_b1, cls_W2, cls_b2):
    raise NotImplementedError("write your pallas kernel here")


# speedup vs baseline: 3.6709x; 3.6709x over previous
import functools

import jax
import jax.numpy as jnp
from jax import lax
from jax.experimental import pallas as pl
from jax.experimental.pallas import tpu as pltpu
from jax.experimental.pallas import tpu_sc as plsc

BN_EPS = 1e-5
_CH = 128
_NC = 2
_NS = 16
_NW = _NC * _NS


def _sc_agg(h, src2d, dst2d, zeros, n_pad):
    _, F = h.shape
    J = src2d.shape[0] // _NW
    rpt = n_pad // _NS
    mesh = plsc.VectorSubcoreMesh(core_axis_name="c", subcore_axis_name="s")

    @functools.partial(
        pl.kernel,
        out_type=jax.ShapeDtypeStruct((_NC, n_pad, F), jnp.float32),
        mesh=mesh,
        scratch_types=[
            pltpu.VMEM((J, _CH), jnp.int32),
            pltpu.VMEM((J, _CH), jnp.int32),
            pltpu.VMEM((_CH, F), jnp.float32),
            pltpu.VMEM_SHARED((n_pad, F), jnp.float32),
            pltpu.SemaphoreType.DMA,
        ],
        compiler_params=pltpu.CompilerParams(use_tc_tiling_on_sc=False),
    )
    def agg_kernel(h_hbm, src_hbm, dst_hbm, z_hbm, out_hbm,
                   src_v, dst_v, rows_v, acc_sh, sem):
        cid = lax.axis_index("c")
        sid = lax.axis_index("s")
        tid = cid * _NS + sid
        pltpu.sync_copy(z_hbm.at[pl.ds(sid * rpt, rpt)],
                        acc_sh.at[pl.ds(sid * rpt, rpt)])
        pltpu.sync_copy(src_hbm.at[pl.ds(tid * J, J)], src_v)
        pltpu.sync_copy(dst_hbm.at[pl.ds(tid * J, J)], dst_v)
        plsc.subcore_barrier()

        def body(j, carry):
            pltpu.async_copy(h_hbm.at[src_v.at[j]], rows_v, sem).wait()
            pltpu.sync_copy(rows_v, acc_sh.at[dst_v.at[j]], add=True)
            return carry

        lax.fori_loop(0, J, body, 0)
        plsc.subcore_barrier()
        pltpu.sync_copy(acc_sh.at[pl.ds(sid * rpt, rpt)],
                        out_hbm.at[cid, pl.ds(sid * rpt, rpt)])

    return agg_kernel(h, src2d, dst2d, zeros)


def _tc_mlp(h, parts, batch3, w1, s1, a1, w2, s2, a2, bn, g):
    n, f = h.shape
    hdim = w1.shape[1]
    grid = (n // bn,)

    def body(h_ref, p_ref, b_ref, w1_ref, s1_ref, a1_ref, w2_ref, s2_ref,
             a2_ref, hout_ref, pool_ref):
        m = h_ref[...] + p_ref[0] + p_ref[1]
        t = jnp.dot(m, w1_ref[...], preferred_element_type=jnp.float32)
        t = jnp.maximum(t * s1_ref[...] + a1_ref[...], 0.0)
        u = jnp.dot(t, w2_ref[...], preferred_element_type=jnp.float32)
        u = jnp.maximum(u * s2_ref[...] + a2_ref[...], 0.0)
        hout_ref[...] = u
        ids = b_ref[0, 0, :]
        oh = (ids[:, None] == lax.broadcasted_iota(jnp.int32, (1, g), 1))
        contrib = lax.dot_general(oh.astype(jnp.float32), u,
                                  (((0,), (0,)), ((), ())),
                                  preferred_element_type=jnp.float32)

        @pl.when(pl.program_id(0) == 0)
        def _():
            pool_ref[...] = jnp.zeros_like(pool_ref)

        pool_ref[...] += contrib

    return pl.pallas_call(
        body,
        grid=grid,
        in_specs=[
            pl.BlockSpec((bn, f), lambda i: (i, 0)),
            pl.BlockSpec((2, bn, f), lambda i: (0, i, 0)),
            pl.BlockSpec((1, 1, bn), lambda i: (i, 0, 0)),
            pl.BlockSpec((f, hdim), lambda i: (0, 0)),
            pl.BlockSpec((1, hdim), lambda i: (0, 0)),
            pl.BlockSpec((1, hdim), lambda i: (0, 0)),
            pl.BlockSpec((hdim, hdim), lambda i: (0, 0)),
            pl.BlockSpec((1, hdim), lambda i: (0, 0)),
            pl.BlockSpec((1, hdim), lambda i: (0, 0)),
        ],
        out_specs=[
            pl.BlockSpec((bn, hdim), lambda i: (i, 0)),
            pl.BlockSpec((g, hdim), lambda i: (0, 0)),
        ],
        out_shape=[
            jax.ShapeDtypeStruct((n, hdim), jnp.float32),
            jax.ShapeDtypeStruct((g, hdim), jnp.float32),
        ],
    )(h, parts, batch3, w1, s1, a1, w2, s2, a2)


def _tc_classifier(p1, p2, p3, w1a, w1b, w1c, b1, w2, b2):
    g, hdim = p1.shape
    c = w2.shape[1]

    def body(p1r, p2r, p3r, w1ar, w1br, w1cr, b1r, w2r, b2r, out_ref):
        t = (jnp.dot(p1r[...], w1ar[...], preferred_element_type=jnp.float32)
             + jnp.dot(p2r[...], w1br[...], preferred_element_type=jnp.float32)
             + jnp.dot(p3r[...], w1cr[...], preferred_element_type=jnp.float32)
             + b1r[...])
        t = jnp.maximum(t, 0.0)
        out_ref[...] = (jnp.dot(t, w2r[...], preferred_element_type=jnp.float32)
                        + b2r[...])

    return pl.pallas_call(
        body,
        out_shape=jax.ShapeDtypeStruct((g, c), jnp.float32),
    )(p1, p2, p3, w1a, w1b, w1c, b1, w2, b2)


def kernel(x, edge_index, batch, mlp_W1, mlp_b1, mlp_g1, mlp_be1, mlp_W2,
           mlp_b2, bn_g, bn_b, cls_W1, cls_b1, cls_W2, cls_b2):
    n, _ = x.shape
    e = edge_index.shape[1]
    g = 64
    layers = len(mlp_W1)

    j_per_tile = -(-(-(-e // (_NW * _CH))) // 8) * 8
    e_pad = _NW * j_per_tile * _CH
    pad = e_pad - e
    src2d = jnp.concatenate(
        [edge_index[0], jnp.zeros((pad,), jnp.int32)]).reshape(-1, _CH)
    dst2d = jnp.concatenate(
        [edge_index[1], jnp.full((pad,), n, jnp.int32)]).reshape(-1, _CH)
    n_pad = -(-(n + 1) // 128) * 128

    bn = 2000
    batch3 = batch.reshape(n // bn, 1, bn)

    sc = 1.0 / jnp.sqrt(jnp.float32(1.0 + BN_EPS))
    h = x
    pooled = []
    for i in range(layers):
        f = h.shape[1]
        zeros = jnp.zeros((n_pad, f), jnp.float32)
        parts = _sc_agg(h, src2d, dst2d, zeros, n_pad)
        s1 = (mlp_g1[i] * sc).reshape(1, -1)
        a1 = (mlp_b1[i] * mlp_g1[i] * sc + mlp_be1[i]).reshape(1, -1)
        s2 = (bn_g[i] * sc).reshape(1, -1)
        a2 = (mlp_b2[i] * bn_g[i] * sc + bn_b[i]).reshape(1, -1)
        h, pool = _tc_mlp(h, parts, batch3, mlp_W1[i], s1, a1, mlp_W2[i],
                          s2, a2, bn, g)
        pooled.append(pool)

    hdim = mlp_W1[0].shape[1]
    w1a, w1b, w1c = (cls_W1[:hdim], cls_W1[hdim:2 * hdim], cls_W1[2 * hdim:])
    return _tc_classifier(pooled[0], pooled[1], pooled[2], w1a, w1b, w1c,
                          cls_b1.reshape(1, -1), cls_W2, cls_b2.reshape(1, -1))


# speedup vs baseline: 4.0058x; 1.0913x over previous
import functools

import jax
import jax.numpy as jnp
from jax import lax
from jax.experimental import pallas as pl
from jax.experimental.pallas import tpu as pltpu
from jax.experimental.pallas import tpu_sc as plsc

BN_EPS = 1e-5
_CH = 128
_NB = 4
_NC = 2
_NS = 16
_NW = _NC * _NS


def _sc_agg(h, src2d, dst2d, zeros, n_pad):
    _, F = h.shape
    J = src2d.shape[0] // _NW
    rpt = n_pad // _NS
    mesh = plsc.VectorSubcoreMesh(core_axis_name="c", subcore_axis_name="s")

    @functools.partial(
        pl.kernel,
        out_type=jax.ShapeDtypeStruct((_NC, n_pad, F), jnp.float32),
        mesh=mesh,
        scratch_types=[
            pltpu.VMEM((J, _CH), jnp.int32),
            pltpu.VMEM((J, _CH), jnp.int32),
            pltpu.VMEM((_NB, _CH, F), jnp.float32),
            pltpu.VMEM_SHARED((n_pad, F), jnp.float32),
        ] + [pltpu.SemaphoreType.DMA] * (2 * _NB),
        compiler_params=pltpu.CompilerParams(use_tc_tiling_on_sc=False),
    )
    def agg_kernel(h_hbm, src_hbm, dst_hbm, z_hbm, out_hbm,
                   src_v, dst_v, rows_v, acc_sh, *sems):
        gsem, ssem = sems[:_NB], sems[_NB:]
        cid = lax.axis_index("c")
        sid = lax.axis_index("s")
        tid = cid * _NS + sid
        pltpu.sync_copy(z_hbm.at[pl.ds(sid * rpt, rpt)],
                        acc_sh.at[pl.ds(sid * rpt, rpt)])
        pltpu.sync_copy(src_hbm.at[pl.ds(tid * J, J)], src_v)
        pltpu.sync_copy(dst_hbm.at[pl.ds(tid * J, J)], dst_v)
        plsc.subcore_barrier()

        def gd(j, b):
            return pltpu.make_async_copy(h_hbm.at[src_v.at[j]],
                                         rows_v.at[b], gsem[b])

        def sd(j, b):
            return pltpu.make_async_copy(rows_v.at[b],
                                         acc_sh.at[dst_v.at[j]], ssem[b])

        ngrp = J // _NB
        for b in range(_NB):
            gd(b, b).start()

        def body(grp, carry):
            for b in range(_NB):
                j = grp * _NB + b
                gd(j, b).wait()
                sd(j, b).start(add=True)
            for b in range(_NB):
                j = grp * _NB + b
                sd(j, b).wait()
                gd(j + _NB, b).start()
            return carry

        lax.fori_loop(0, ngrp - 1, body, 0)
        for b in range(_NB):
            j = (ngrp - 1) * _NB + b
            gd(j, b).wait()
            sd(j, b).start(add=True)
        for b in range(_NB):
            sd((ngrp - 1) * _NB + b, b).wait()
        plsc.subcore_barrier()
        pltpu.sync_copy(acc_sh.at[pl.ds(sid * rpt, rpt)],
                        out_hbm.at[cid, pl.ds(sid * rpt, rpt)])

    return agg_kernel(h, src2d, dst2d, zeros)


def _tc_layer(hs, parts_list, w1, batch3, s1, a1, w2, s2, a2, bn, g):
    k = len(hs)
    n, hdim = hs[0].shape
    grid = (n // bn,)

    def body(*refs):
        h_refs = refs[:k]
        p_refs = refs[k:2 * k]
        w1_ref = refs[2 * k]
        b_ref, s1_ref, a1_ref, w2_ref, s2_ref, a2_ref, u_ref, pool_ref = \
            refs[2 * k + 1:]
        ms = [h_ref[...] + p_ref[0] + p_ref[1]
              for h_ref, p_ref in zip(h_refs, p_refs)]
        m = ms[0] if k == 1 else jnp.concatenate(ms, axis=1)
        acc = jnp.dot(m, w1_ref[...], preferred_element_type=jnp.float32)
        t = jnp.maximum(acc * s1_ref[...] + a1_ref[...], 0.0)
        u = jnp.dot(t, w2_ref[...], preferred_element_type=jnp.float32)
        u = jnp.maximum(u * s2_ref[...] + a2_ref[...], 0.0)
        u_ref[...] = u
        ids = b_ref[0, 0, :]
        oh = (ids[:, None] == lax.broadcasted_iota(jnp.int32, (1, g), 1))
        contrib = lax.dot_general(oh.astype(jnp.float32), u,
                                  (((0,), (0,)), ((), ())),
                                  preferred_element_type=jnp.float32)

        @pl.when(pl.program_id(0) == 0)
        def _():
            pool_ref[...] = jnp.zeros_like(pool_ref)

        pool_ref[...] += contrib

    vec = lambda: pl.BlockSpec((1, hdim), lambda i: (0, 0))
    in_specs = (
        [pl.BlockSpec((bn, hdim), lambda i: (i, 0))] * k
        + [pl.BlockSpec((2, bn, hdim), lambda i: (0, i, 0))] * k
        + [pl.BlockSpec((k * hdim, hdim), lambda i: (0, 0))]
        + [pl.BlockSpec((1, 1, bn), lambda i: (i, 0, 0)),
           vec(), vec(),
           pl.BlockSpec((hdim, hdim), lambda i: (0, 0)),
           vec(), vec()]
    )
    return pl.pallas_call(
        body,
        grid=grid,
        in_specs=in_specs,
        out_specs=[
            pl.BlockSpec((bn, hdim), lambda i: (i, 0)),
            pl.BlockSpec((g, hdim), lambda i: (0, 0)),
        ],
        out_shape=[
            jax.ShapeDtypeStruct((n, hdim), jnp.float32),
            jax.ShapeDtypeStruct((g, hdim), jnp.float32),
        ],
    )(*hs, *parts_list, w1, batch3, s1, a1, w2, s2, a2)


def _tc_classifier(p1, p2, p3, w1a, w1b, w1c, b1, w2, b2):
    g, hdim = p1.shape
    c = w2.shape[1]

    def body(p1r, p2r, p3r, w1ar, w1br, w1cr, b1r, w2r, b2r, out_ref):
        t = (jnp.dot(p1r[...], w1ar[...], preferred_element_type=jnp.float32)
             + jnp.dot(p2r[...], w1br[...], preferred_element_type=jnp.float32)
             + jnp.dot(p3r[...], w1cr[...], preferred_element_type=jnp.float32)
             + b1r[...])
        t = jnp.maximum(t, 0.0)
        out_ref[...] = (jnp.dot(t, w2r[...], preferred_element_type=jnp.float32)
                        + b2r[...])

    return pl.pallas_call(
        body,
        out_shape=jax.ShapeDtypeStruct((g, c), jnp.float32),
    )(p1, p2, p3, w1a, w1b, w1c, b1, w2, b2)


def kernel(x, edge_index, batch, mlp_W1, mlp_b1, mlp_g1, mlp_be1, mlp_W2,
           mlp_b2, bn_g, bn_b, cls_W1, cls_b1, cls_W2, cls_b2):
    n, f_in = x.shape
    e = edge_index.shape[1]
    g = 64
    layers = len(mlp_W1)
    hdim = mlp_W1[0].shape[1]

    j_per_tile = -(-(-(-e // (_NW * _CH))) // 8) * 8
    e_pad = _NW * j_per_tile * _CH
    pad = e_pad - e
    src2d = jnp.concatenate(
        [edge_index[0], jnp.zeros((pad,), jnp.int32)]).reshape(-1, _CH)
    dst2d = jnp.concatenate(
        [edge_index[1], jnp.full((pad,), n, jnp.int32)]).reshape(-1, _CH)
    n_pad = -(-(n + 1) // 128) * 128
    zeros = jnp.zeros((n_pad, hdim), jnp.float32)

    bn = 2000
    batch3 = batch.reshape(n // bn, 1, bn)

    sc = 1.0 / jnp.sqrt(jnp.float32(1.0 + BN_EPS))
    h = x
    pooled = []
    for i in range(layers):
        f = h.shape[1]
        if f == hdim:
            hs = [h]
        else:
            hs = [h[:, j * hdim:(j + 1) * hdim] for j in range(f // hdim)]
        parts_list = [_sc_agg(hh, src2d, dst2d, zeros, n_pad) for hh in hs]
        s1 = (mlp_g1[i] * sc).reshape(1, -1)
        a1 = (mlp_b1[i] * mlp_g1[i] * sc + mlp_be1[i]).reshape(1, -1)
        s2 = (bn_g[i] * sc).reshape(1, -1)
        a2 = (mlp_b2[i] * bn_g[i] * sc + bn_b[i]).reshape(1, -1)
        h, pool = _tc_layer(hs, parts_list, mlp_W1[i], batch3, s1, a1,
                            mlp_W2[i], s2, a2, bn, g)
        pooled.append(pool)

    w1a, w1b, w1c = (cls_W1[:hdim], cls_W1[hdim:2 * hdim], cls_W1[2 * hdim:])
    return _tc_classifier(pooled[0], pooled[1], pooled[2], w1a, w1b, w1c,
                          cls_b1.reshape(1, -1), cls_W2, cls_b2.reshape(1, -1))


# speedup vs baseline: 4.0763x; 1.0176x over previous
import functools

import jax
import jax.numpy as jnp
from jax import lax
from jax.experimental import pallas as pl
from jax.experimental.pallas import tpu as pltpu
from jax.experimental.pallas import tpu_sc as plsc

BN_EPS = 1e-5
_CH = 128
_NB = 8
_NC = 2
_NS = 16
_NW = _NC * _NS


def _sc_agg(h, src2d, dst2d, zeros, n_pad):
    _, F = h.shape
    J = src2d.shape[0] // _NW
    rpt = n_pad // _NS
    mesh = plsc.VectorSubcoreMesh(core_axis_name="c", subcore_axis_name="s")

    @functools.partial(
        pl.kernel,
        out_type=jax.ShapeDtypeStruct((_NC, n_pad, F), jnp.float32),
        mesh=mesh,
        scratch_types=[
            pltpu.VMEM((J, _CH), jnp.int32),
            pltpu.VMEM((J, _CH), jnp.int32),
            pltpu.VMEM((_NB, _CH, F), jnp.float32),
            pltpu.VMEM_SHARED((n_pad, F), jnp.float32),
        ] + [pltpu.SemaphoreType.DMA] * (2 * _NB),
        compiler_params=pltpu.CompilerParams(use_tc_tiling_on_sc=False),
    )
    def agg_kernel(h_hbm, src_hbm, dst_hbm, z_hbm, out_hbm,
                   src_v, dst_v, rows_v, acc_sh, *sems):
        gsem, ssem = sems[:_NB], sems[_NB:]
        cid = lax.axis_index("c")
        sid = lax.axis_index("s")
        tid = cid * _NS + sid
        pltpu.sync_copy(z_hbm.at[pl.ds(sid * rpt, rpt)],
                        acc_sh.at[pl.ds(sid * rpt, rpt)])
        pltpu.sync_copy(src_hbm.at[pl.ds(tid * J, J)], src_v)
        pltpu.sync_copy(dst_hbm.at[pl.ds(tid * J, J)], dst_v)
        plsc.subcore_barrier()

        def gd(j, b):
            return pltpu.make_async_copy(h_hbm.at[src_v.at[j]],
                                         rows_v.at[b], gsem[b])

        def sd(j, b):
            return pltpu.make_async_copy(rows_v.at[b],
                                         acc_sh.at[dst_v.at[j]], ssem[b])

        ngrp = J // _NB
        for b in range(_NB):
            gd(b, b).start()

        def body(grp, carry):
            for b in range(_NB):
                j = grp * _NB + b
                gd(j, b).wait()
                sd(j, b).start(add=True)
            for b in range(_NB):
                j = grp * _NB + b
                sd(j, b).wait()
                gd(j + _NB, b).start()
            return carry

        lax.fori_loop(0, ngrp - 1, body, 0)
        for b in range(_NB):
            j = (ngrp - 1) * _NB + b
            gd(j, b).wait()
            sd(j, b).start(add=True)
        for b in range(_NB):
            sd((ngrp - 1) * _NB + b, b).wait()
        plsc.subcore_barrier()
        pltpu.sync_copy(acc_sh.at[pl.ds(sid * rpt, rpt)],
                        out_hbm.at[cid, pl.ds(sid * rpt, rpt)])

    return agg_kernel(h, src2d, dst2d, zeros)


def _tc_layer(hs, parts_list, w1, batch3, s1, a1, w2, s2, a2, bn, g):
    k = len(hs)
    n, hdim = hs[0].shape
    grid = (n // bn,)

    def body(*refs):
        h_refs = refs[:k]
        p_refs = refs[k:2 * k]
        w1_ref = refs[2 * k]
        b_ref, s1_ref, a1_ref, w2_ref, s2_ref, a2_ref, u_ref, pool_ref = \
            refs[2 * k + 1:]
        ms = [h_ref[...] + p_ref[0] + p_ref[1]
              for h_ref, p_ref in zip(h_refs, p_refs)]
        m = ms[0] if k == 1 else jnp.concatenate(ms, axis=1)
        acc = jnp.dot(m, w1_ref[...], preferred_element_type=jnp.float32)
        t = jnp.maximum(acc * s1_ref[...] + a1_ref[...], 0.0)
        u = jnp.dot(t, w2_ref[...], preferred_element_type=jnp.float32)
        u = jnp.maximum(u * s2_ref[...] + a2_ref[...], 0.0)
        u_ref[...] = u
        ids = b_ref[0, 0, :]
        oh = (ids[:, None] == lax.broadcasted_iota(jnp.int32, (1, g), 1))
        contrib = lax.dot_general(oh.astype(jnp.float32), u,
                                  (((0,), (0,)), ((), ())),
                                  preferred_element_type=jnp.float32)

        @pl.when(pl.program_id(0) == 0)
        def _():
            pool_ref[...] = jnp.zeros_like(pool_ref)

        pool_ref[...] += contrib

    vec = lambda: pl.BlockSpec((1, hdim), lambda i: (0, 0))
    in_specs = (
        [pl.BlockSpec((bn, hdim), lambda i: (i, 0))] * k
        + [pl.BlockSpec((2, bn, hdim), lambda i: (0, i, 0))] * k
        + [pl.BlockSpec((k * hdim, hdim), lambda i: (0, 0))]
        + [pl.BlockSpec((1, 1, bn), lambda i: (i, 0, 0)),
           vec(), vec(),
           pl.BlockSpec((hdim, hdim), lambda i: (0, 0)),
           vec(), vec()]
    )
    return pl.pallas_call(
        body,
        grid=grid,
        in_specs=in_specs,
        out_specs=[
            pl.BlockSpec((bn, hdim), lambda i: (i, 0)),
            pl.BlockSpec((g, hdim), lambda i: (0, 0)),
        ],
        out_shape=[
            jax.ShapeDtypeStruct((n, hdim), jnp.float32),
            jax.ShapeDtypeStruct((g, hdim), jnp.float32),
        ],
    )(*hs, *parts_list, w1, batch3, s1, a1, w2, s2, a2)


def _tc_classifier(p1, p2, p3, w1a, w1b, w1c, b1, w2, b2):
    g, hdim = p1.shape
    c = w2.shape[1]

    def body(p1r, p2r, p3r, w1ar, w1br, w1cr, b1r, w2r, b2r, out_ref):
        t = (jnp.dot(p1r[...], w1ar[...], preferred_element_type=jnp.float32)
             + jnp.dot(p2r[...], w1br[...], preferred_element_type=jnp.float32)
             + jnp.dot(p3r[...], w1cr[...], preferred_element_type=jnp.float32)
             + b1r[...])
        t = jnp.maximum(t, 0.0)
        out_ref[...] = (jnp.dot(t, w2r[...], preferred_element_type=jnp.float32)
                        + b2r[...])

    return pl.pallas_call(
        body,
        out_shape=jax.ShapeDtypeStruct((g, c), jnp.float32),
    )(p1, p2, p3, w1a, w1b, w1c, b1, w2, b2)


def kernel(x, edge_index, batch, mlp_W1, mlp_b1, mlp_g1, mlp_be1, mlp_W2,
           mlp_b2, bn_g, bn_b, cls_W1, cls_b1, cls_W2, cls_b2):
    n, f_in = x.shape
    e = edge_index.shape[1]
    g = 64
    layers = len(mlp_W1)
    hdim = mlp_W1[0].shape[1]

    j_per_tile = -(-(-(-e // (_NW * _CH))) // 8) * 8
    e_pad = _NW * j_per_tile * _CH
    pad = e_pad - e
    src2d = jnp.concatenate(
        [edge_index[0], jnp.zeros((pad,), jnp.int32)]).reshape(-1, _CH)
    dst2d = jnp.concatenate(
        [edge_index[1], jnp.full((pad,), n, jnp.int32)]).reshape(-1, _CH)
    n_pad = -(-(n + 1) // 128) * 128
    zeros = jnp.zeros((n_pad, hdim), jnp.float32)

    bn = 2000
    batch3 = batch.reshape(n // bn, 1, bn)

    sc = 1.0 / jnp.sqrt(jnp.float32(1.0 + BN_EPS))
    h = x
    pooled = []
    for i in range(layers):
        f = h.shape[1]
        if f == hdim:
            hs = [h]
        else:
            hs = [h[:, j * hdim:(j + 1) * hdim] for j in range(f // hdim)]
        parts_list = [_sc_agg(hh, src2d, dst2d, zeros, n_pad) for hh in hs]
        s1 = (mlp_g1[i] * sc).reshape(1, -1)
        a1 = (mlp_b1[i] * mlp_g1[i] * sc + mlp_be1[i]).reshape(1, -1)
        s2 = (bn_g[i] * sc).reshape(1, -1)
        a2 = (mlp_b2[i] * bn_g[i] * sc + bn_b[i]).reshape(1, -1)
        h, pool = _tc_layer(hs, parts_list, mlp_W1[i], batch3, s1, a1,
                            mlp_W2[i], s2, a2, bn, g)
        pooled.append(pool)

    w1a, w1b, w1c = (cls_W1[:hdim], cls_W1[hdim:2 * hdim], cls_W1[2 * hdim:])
    return _tc_classifier(pooled[0], pooled[1], pooled[2], w1a, w1b, w1c,
                          cls_b1.reshape(1, -1), cls_W2, cls_b2.reshape(1, -1))


# speedup vs baseline: 4.3376x; 1.0641x over previous
import functools

import jax
import jax.numpy as jnp
from jax import lax
from jax.experimental import pallas as pl
from jax.experimental.pallas import tpu as pltpu
from jax.experimental.pallas import tpu_sc as plsc

BN_EPS = 1e-5
_CH = 128
_NB = 4
_NC = 2
_NS = 16
_NW = _NC * _NS


def _sc_agg(h, src2d, dst2d, zeros, n_pad, j0, j1):
    _, F = h.shape
    jmax = max(j0, j1)
    rpt = n_pad // _NS
    mesh = plsc.VectorSubcoreMesh(core_axis_name="c", subcore_axis_name="s")

    @functools.partial(
        pl.kernel,
        out_type=jax.ShapeDtypeStruct((_NC, n_pad, F), jnp.float32),
        mesh=mesh,
        scratch_types=[
            pltpu.VMEM((jmax, _CH), jnp.int32),
            pltpu.VMEM((jmax, _CH), jnp.int32),
            pltpu.VMEM((_NB, _CH, F), jnp.float32),
            pltpu.VMEM_SHARED((n_pad, F), jnp.float32),
        ] + [pltpu.SemaphoreType.DMA] * (2 * _NB),
        compiler_params=pltpu.CompilerParams(use_tc_tiling_on_sc=False),
    )
    def agg_kernel(h_hbm, src_hbm, dst_hbm, z_hbm, out_hbm,
                   src_v, dst_v, rows_v, acc_sh, *sems):
        gsem, ssem = sems[:_NB], sems[_NB:]
        cid = lax.axis_index("c")
        sid = lax.axis_index("s")
        pltpu.sync_copy(z_hbm.at[pl.ds(sid * rpt, rpt)],
                        acc_sh.at[pl.ds(sid * rpt, rpt)])
        plsc.subcore_barrier()

        def gd(j, b):
            return pltpu.make_async_copy(h_hbm.at[src_v.at[j]],
                                         rows_v.at[b], gsem[b])

        def sd(j, b):
            return pltpu.make_async_copy(rows_v.at[b],
                                         acc_sh.at[dst_v.at[j]], ssem[b])

        def run_side(jc, base):
            pltpu.sync_copy(src_hbm.at[pl.ds(base, jc)],
                            src_v.at[pl.ds(0, jc)])
            pltpu.sync_copy(dst_hbm.at[pl.ds(base, jc)],
                            dst_v.at[pl.ds(0, jc)])
            ngrp = jc // _NB
            for b in range(_NB):
                gd(b, b).start()

            def body(grp, carry):
                for b in range(_NB):
                    j = grp * _NB + b
                    gd(j, b).wait()
                    sd(j, b).start(add=True)
                for b in range(_NB):
                    j = grp * _NB + b
                    sd(j, b).wait()
                    gd(j + _NB, b).start()
                return carry

            lax.fori_loop(0, ngrp - 1, body, 0)
            for b in range(_NB):
                j = (ngrp - 1) * _NB + b
                gd(j, b).wait()
                sd(j, b).start(add=True)
            for b in range(_NB):
                sd((ngrp - 1) * _NB + b, b).wait()

        @pl.when(cid == 0)
        def _():
            run_side(j0, sid * j0)

        @pl.when(cid == 1)
        def _():
            run_side(j1, _NS * j0 + sid * j1)

        plsc.subcore_barrier()
        pltpu.sync_copy(acc_sh.at[pl.ds(sid * rpt, rpt)],
                        out_hbm.at[cid, pl.ds(sid * rpt, rpt)])

    return agg_kernel(h, src2d, dst2d, zeros)


def _tc_layer(hs, parts_list, w1, batch3, s1, a1, w2, s2, a2, bn, g):
    k = len(hs)
    n, hdim = hs[0].shape
    grid = (n // bn,)

    def body(*refs):
        h_refs = refs[:k]
        p_refs = refs[k:2 * k]
        w1_ref = refs[2 * k]
        b_ref, s1_ref, a1_ref, w2_ref, s2_ref, a2_ref, u_ref, pool_ref = \
            refs[2 * k + 1:]
        ms = [h_ref[...] + p_ref[0] + p_ref[1]
              for h_ref, p_ref in zip(h_refs, p_refs)]
        m = ms[0] if k == 1 else jnp.concatenate(ms, axis=1)
        acc = jnp.dot(m, w1_ref[...], preferred_element_type=jnp.float32)
        t = jnp.maximum(acc * s1_ref[...] + a1_ref[...], 0.0)
        u = jnp.dot(t, w2_ref[...], preferred_element_type=jnp.float32)
        u = jnp.maximum(u * s2_ref[...] + a2_ref[...], 0.0)
        u_ref[...] = u
        ids = b_ref[0, 0, :]
        oh = (ids[:, None] == lax.broadcasted_iota(jnp.int32, (1, g), 1))
        contrib = lax.dot_general(oh.astype(jnp.float32), u,
                                  (((0,), (0,)), ((), ())),
                                  preferred_element_type=jnp.float32)

        @pl.when(pl.program_id(0) == 0)
        def _():
            pool_ref[...] = jnp.zeros_like(pool_ref)

        pool_ref[...] += contrib

    vec = lambda: pl.BlockSpec((1, hdim), lambda i: (0, 0))
    in_specs = (
        [pl.BlockSpec((bn, hdim), lambda i: (i, 0))] * k
        + [pl.BlockSpec((2, bn, hdim), lambda i: (0, i, 0))] * k
        + [pl.BlockSpec((k * hdim, hdim), lambda i: (0, 0))]
        + [pl.BlockSpec((1, 1, bn), lambda i: (i, 0, 0)),
           vec(), vec(),
           pl.BlockSpec((hdim, hdim), lambda i: (0, 0)),
           vec(), vec()]
    )
    return pl.pallas_call(
        body,
        grid=grid,
        in_specs=in_specs,
        out_specs=[
            pl.BlockSpec((bn, hdim), lambda i: (i, 0)),
            pl.BlockSpec((g, hdim), lambda i: (0, 0)),
        ],
        out_shape=[
            jax.ShapeDtypeStruct((n, hdim), jnp.float32),
            jax.ShapeDtypeStruct((g, hdim), jnp.float32),
        ],
    )(*hs, *parts_list, w1, batch3, s1, a1, w2, s2, a2)


def _tc_classifier(p1, p2, p3, w1a, w1b, w1c, b1, w2, b2):
    g, hdim = p1.shape
    c = w2.shape[1]

    def body(p1r, p2r, p3r, w1ar, w1br, w1cr, b1r, w2r, b2r, out_ref):
        t = (jnp.dot(p1r[...], w1ar[...], preferred_element_type=jnp.float32)
             + jnp.dot(p2r[...], w1br[...], preferred_element_type=jnp.float32)
             + jnp.dot(p3r[...], w1cr[...], preferred_element_type=jnp.float32)
             + b1r[...])
        t = jnp.maximum(t, 0.0)
        out_ref[...] = (jnp.dot(t, w2r[...], preferred_element_type=jnp.float32)
                        + b2r[...])

    return pl.pallas_call(
        body,
        out_shape=jax.ShapeDtypeStruct((g, c), jnp.float32),
    )(p1, p2, p3, w1a, w1b, w1c, b1, w2, b2)


def kernel(x, edge_index, batch, mlp_W1, mlp_b1, mlp_g1, mlp_be1, mlp_W2,
           mlp_b2, bn_g, bn_b, cls_W1, cls_b1, cls_W2, cls_b2):
    n, f_in = x.shape
    e = edge_index.shape[1]
    g = 64
    layers = len(mlp_W1)
    hdim = mlp_W1[0].shape[1]

    j_per_tile = -(-(-(-e // (_NW * _CH))) // 8) * 8
    e_pad = _NW * j_per_tile * _CH
    j0 = (2 * j_per_tile * 4 // 5) // 8 * 8
    j1 = 2 * j_per_tile - j0
    pad = e_pad - e
    src2d = jnp.concatenate(
        [edge_index[0], jnp.zeros((pad,), jnp.int32)]).reshape(-1, _CH)
    dst2d = jnp.concatenate(
        [edge_index[1], jnp.full((pad,), n, jnp.int32)]).reshape(-1, _CH)
    n_pad = -(-(n + 1) // 128) * 128
    zeros = jnp.zeros((n_pad, hdim), jnp.float32)

    bn = 2000
    batch3 = batch.reshape(n // bn, 1, bn)

    sc = 1.0 / jnp.sqrt(jnp.float32(1.0 + BN_EPS))
    h = x
    pooled = []
    for i in range(layers):
        f = h.shape[1]
        if f == hdim:
            hs = [h]
        else:
            hs = [h[:, j * hdim:(j + 1) * hdim] for j in range(f // hdim)]
        parts_list = [_sc_agg(hh, src2d, dst2d, zeros, n_pad, j0, j1)
                      for hh in hs]
        s1 = (mlp_g1[i] * sc).reshape(1, -1)
        a1 = (mlp_b1[i] * mlp_g1[i] * sc + mlp_be1[i]).reshape(1, -1)
        s2 = (bn_g[i] * sc).reshape(1, -1)
        a2 = (mlp_b2[i] * bn_g[i] * sc + bn_b[i]).reshape(1, -1)
        h, pool = _tc_layer(hs, parts_list, mlp_W1[i], batch3, s1, a1,
                            mlp_W2[i], s2, a2, bn, g)
        pooled.append(pool)

    w1a, w1b, w1c = (cls_W1[:hdim], cls_W1[hdim:2 * hdim], cls_W1[2 * hdim:])
    return _tc_classifier(pooled[0], pooled[1], pooled[2], w1a, w1b, w1c,
                          cls_b1.reshape(1, -1), cls_W2, cls_b2.reshape(1, -1))


# speedup vs baseline: 4.9137x; 1.1328x over previous
import functools

import jax
import jax.numpy as jnp
from jax import lax
from jax.experimental import pallas as pl
from jax.experimental.pallas import tpu as pltpu
from jax.experimental.pallas import tpu_sc as plsc

BN_EPS = 1e-5
_CH = 128
_NB = 4
_NC = 2
_NS = 16
_NW = _NC * _NS


def _sc_agg(h, src2d, dst2d, zeros, n_pad, j0, j1):
    _, F = h.shape
    jmax = max(j0, j1)
    rpt = n_pad // _NS
    mesh = plsc.VectorSubcoreMesh(core_axis_name="c", subcore_axis_name="s")

    @functools.partial(
        pl.kernel,
        out_type=jax.ShapeDtypeStruct((_NC, n_pad, F), jnp.float32),
        mesh=mesh,
        scratch_types=[
            pltpu.VMEM((jmax, _CH), jnp.int32),
            pltpu.VMEM((jmax, _CH), jnp.int32),
            pltpu.VMEM((_NB, _CH, F), jnp.float32),
            pltpu.VMEM_SHARED((n_pad, F), jnp.float32),
        ] + [pltpu.SemaphoreType.DMA] * (2 * _NB),
        compiler_params=pltpu.CompilerParams(use_tc_tiling_on_sc=False),
    )
    def agg_kernel(h_hbm, src_hbm, dst_hbm, z_hbm, out_hbm,
                   src_v, dst_v, rows_v, acc_sh, *sems):
        gsem, ssem = sems[:_NB], sems[_NB:]
        cid = lax.axis_index("c")
        sid = lax.axis_index("s")
        pltpu.sync_copy(z_hbm.at[pl.ds(sid * rpt, rpt)],
                        acc_sh.at[pl.ds(sid * rpt, rpt)])
        plsc.subcore_barrier()

        def gd(j, b):
            return pltpu.make_async_copy(h_hbm.at[src_v.at[j]],
                                         rows_v.at[b], gsem[b])

        def sd(j, b):
            return pltpu.make_async_copy(rows_v.at[b],
                                         acc_sh.at[dst_v.at[j]], ssem[b])

        def run_side(jc, base):
            pltpu.sync_copy(src_hbm.at[pl.ds(base, jc)],
                            src_v.at[pl.ds(0, jc)])
            pltpu.sync_copy(dst_hbm.at[pl.ds(base, jc)],
                            dst_v.at[pl.ds(0, jc)])
            ngrp = jc // _NB
            for b in range(_NB):
                gd(b, b).start()

            def body(grp, carry):
                for b in range(_NB):
                    j = grp * _NB + b
                    gd(j, b).wait()
                    sd(j, b).start(add=True)
                for b in range(_NB):
                    j = grp * _NB + b
                    sd(j, b).wait()
                    gd(j + _NB, b).start()
                return carry

            lax.fori_loop(0, ngrp - 1, body, 0)
            for b in range(_NB):
                j = (ngrp - 1) * _NB + b
                gd(j, b).wait()
                sd(j, b).start(add=True)
            for b in range(_NB):
                sd((ngrp - 1) * _NB + b, b).wait()

        @pl.when(cid == 0)
        def _():
            run_side(j0, sid * j0)

        @pl.when(cid == 1)
        def _():
            run_side(j1, _NS * j0 + sid * j1)

        plsc.subcore_barrier()
        pltpu.sync_copy(acc_sh.at[pl.ds(sid * rpt, rpt)],
                        out_hbm.at[cid, pl.ds(sid * rpt, rpt)])

    return agg_kernel(h, src2d, dst2d, zeros)


def _tc_layer(hs, parts_list, w1, batch3, s1, a1, w2, s2, a2, bn, g):
    k = len(hs)
    n, hdim = hs[0].shape
    grid = (n // bn,)

    def body(*refs):
        h_refs = refs[:k]
        p_refs = refs[k:2 * k]
        w1_ref = refs[2 * k]
        b_ref, s1_ref, a1_ref, w2_ref, s2_ref, a2_ref, u_ref, pool_ref = \
            refs[2 * k + 1:]
        ms = [h_ref[...] + p_ref[0] + p_ref[1]
              for h_ref, p_ref in zip(h_refs, p_refs)]
        m = ms[0] if k == 1 else jnp.concatenate(ms, axis=1)
        acc = jnp.dot(m, w1_ref[...], preferred_element_type=jnp.float32)
        t = jnp.maximum(acc * s1_ref[...] + a1_ref[...], 0.0)
        u = jnp.dot(t, w2_ref[...], preferred_element_type=jnp.float32)
        u = jnp.maximum(u * s2_ref[...] + a2_ref[...], 0.0)
        u_ref[...] = u
        ids = b_ref[0, 0, :]
        oh = (ids[:, None] == lax.broadcasted_iota(jnp.int32, (1, g), 1))
        contrib = lax.dot_general(oh.astype(jnp.float32), u,
                                  (((0,), (0,)), ((), ())),
                                  preferred_element_type=jnp.float32)

        @pl.when(pl.program_id(0) == 0)
        def _():
            pool_ref[...] = jnp.zeros_like(pool_ref)

        pool_ref[...] += contrib

    vec = lambda: pl.BlockSpec((1, hdim), lambda i: (0, 0))
    in_specs = (
        [pl.BlockSpec((bn, hdim), lambda i: (i, 0))] * k
        + [pl.BlockSpec((2, bn, hdim), lambda i: (0, i, 0))] * k
        + [pl.BlockSpec((k * hdim, hdim), lambda i: (0, 0))]
        + [pl.BlockSpec((1, 1, bn), lambda i: (i, 0, 0)),
           vec(), vec(),
           pl.BlockSpec((hdim, hdim), lambda i: (0, 0)),
           vec(), vec()]
    )
    return pl.pallas_call(
        body,
        grid=grid,
        in_specs=in_specs,
        out_specs=[
            pl.BlockSpec((bn, hdim), lambda i: (i, 0)),
            pl.BlockSpec((g, hdim), lambda i: (0, 0)),
        ],
        out_shape=[
            jax.ShapeDtypeStruct((n, hdim), jnp.float32),
            jax.ShapeDtypeStruct((g, hdim), jnp.float32),
        ],
    )(*hs, *parts_list, w1, batch3, s1, a1, w2, s2, a2)


def _tc_classifier(p1, p2, p3, w1a, w1b, w1c, b1, w2, b2):
    g, hdim = p1.shape
    c = w2.shape[1]

    def body(p1r, p2r, p3r, w1ar, w1br, w1cr, b1r, w2r, b2r, out_ref):
        t = (jnp.dot(p1r[...], w1ar[...], preferred_element_type=jnp.float32)
             + jnp.dot(p2r[...], w1br[...], preferred_element_type=jnp.float32)
             + jnp.dot(p3r[...], w1cr[...], preferred_element_type=jnp.float32)
             + b1r[...])
        t = jnp.maximum(t, 0.0)
        out_ref[...] = (jnp.dot(t, w2r[...], preferred_element_type=jnp.float32)
                        + b2r[...])

    return pl.pallas_call(
        body,
        out_shape=jax.ShapeDtypeStruct((g, c), jnp.float32),
    )(p1, p2, p3, w1a, w1b, w1c, b1, w2, b2)


def kernel(x, edge_index, batch, mlp_W1, mlp_b1, mlp_g1, mlp_be1, mlp_W2,
           mlp_b2, bn_g, bn_b, cls_W1, cls_b1, cls_W2, cls_b2):
    n, f_in = x.shape
    e = edge_index.shape[1]
    g = 64
    layers = len(mlp_W1)
    hdim = mlp_W1[0].shape[1]

    j_per_tile = -(-(-(-e // (_NW * _CH))) // 8) * 8
    e_pad = _NW * j_per_tile * _CH
    j0 = (2 * j_per_tile * 9 // 10) // 8 * 8
    j1 = 2 * j_per_tile - j0
    pad = e_pad - e
    src2d = jnp.concatenate(
        [edge_index[0], jnp.zeros((pad,), jnp.int32)]).reshape(-1, _CH)
    dst2d = jnp.concatenate(
        [edge_index[1], jnp.full((pad,), n, jnp.int32)]).reshape(-1, _CH)
    n_pad = -(-(n + 1) // 128) * 128
    zeros = jnp.zeros((n_pad, hdim), jnp.float32)

    bn = 2000
    batch3 = batch.reshape(n // bn, 1, bn)

    sc = 1.0 / jnp.sqrt(jnp.float32(1.0 + BN_EPS))
    h = x
    pooled = []
    for i in range(layers):
        f = h.shape[1]
        if f == hdim:
            hs = [h]
        else:
            hs = [h[:, j * hdim:(j + 1) * hdim] for j in range(f // hdim)]
        parts_list = [_sc_agg(hh, src2d, dst2d, zeros, n_pad, j0, j1)
                      for hh in hs]
        s1 = (mlp_g1[i] * sc).reshape(1, -1)
        a1 = (mlp_b1[i] * mlp_g1[i] * sc + mlp_be1[i]).reshape(1, -1)
        s2 = (bn_g[i] * sc).reshape(1, -1)
        a2 = (mlp_b2[i] * bn_g[i] * sc + bn_b[i]).reshape(1, -1)
        h, pool = _tc_layer(hs, parts_list, mlp_W1[i], batch3, s1, a1,
                            mlp_W2[i], s2, a2, bn, g)
        pooled.append(pool)

    w1a, w1b, w1c = (cls_W1[:hdim], cls_W1[hdim:2 * hdim], cls_W1[2 * hdim:])
    return _tc_classifier(pooled[0], pooled[1], pooled[2], w1a, w1b, w1c,
                          cls_b1.reshape(1, -1), cls_W2, cls_b2.reshape(1, -1))


# speedup vs baseline: 4.9548x; 1.0084x over previous
import functools

import jax
import jax.numpy as jnp
from jax import lax
from jax.experimental import pallas as pl
from jax.experimental.pallas import tpu as pltpu
from jax.experimental.pallas import tpu_sc as plsc

BN_EPS = 1e-5
_CH = 128
_NB = 4
_NC = 2
_NS = 16
_NW = _NC * _NS


def _sc_agg(h, src2d, dst2d, zeros, n_pad, j0, j1):
    _, F = h.shape
    jmax = max(j0, j1)
    rpt = n_pad // _NS
    mesh = plsc.VectorSubcoreMesh(core_axis_name="c", subcore_axis_name="s")

    @functools.partial(
        pl.kernel,
        out_type=jax.ShapeDtypeStruct((_NC, n_pad, F), jnp.float32),
        mesh=mesh,
        scratch_types=[
            pltpu.VMEM((jmax, _CH), jnp.int32),
            pltpu.VMEM((jmax, _CH), jnp.int32),
            pltpu.VMEM((_NB, _CH, F), jnp.float32),
            pltpu.VMEM_SHARED((n_pad, F), jnp.float32),
        ] + [pltpu.SemaphoreType.DMA] * (2 * _NB),
        compiler_params=pltpu.CompilerParams(use_tc_tiling_on_sc=False),
    )
    def agg_kernel(h_hbm, src_hbm, dst_hbm, z_hbm, out_hbm,
                   src_v, dst_v, rows_v, acc_sh, *sems):
        gsem, ssem = sems[:_NB], sems[_NB:]
        cid = lax.axis_index("c")
        sid = lax.axis_index("s")
        pltpu.sync_copy(z_hbm.at[pl.ds(sid * rpt, rpt)],
                        acc_sh.at[pl.ds(sid * rpt, rpt)])
        plsc.subcore_barrier()

        def gd(j, b):
            return pltpu.make_async_copy(h_hbm.at[src_v.at[j]],
                                         rows_v.at[b], gsem[b])

        def sd(j, b):
            return pltpu.make_async_copy(rows_v.at[b],
                                         acc_sh.at[dst_v.at[j]], ssem[b])

        def run_side(jc, base):
            pltpu.sync_copy(src_hbm.at[pl.ds(base, jc)],
                            src_v.at[pl.ds(0, jc)])
            pltpu.sync_copy(dst_hbm.at[pl.ds(base, jc)],
                            dst_v.at[pl.ds(0, jc)])
            ngrp = jc // _NB
            for b in range(_NB):
                gd(b, b).start()

            def body(grp, carry):
                for b in range(_NB):
                    j = grp * _NB + b
                    gd(j, b).wait()
                    sd(j, b).start(add=True)
                for b in range(_NB):
                    j = grp * _NB + b
                    sd(j, b).wait()
                    gd(j + _NB, b).start()
                return carry

            lax.fori_loop(0, ngrp - 1, body, 0)
            for b in range(_NB):
                j = (ngrp - 1) * _NB + b
                gd(j, b).wait()
                sd(j, b).start(add=True)
            for b in range(_NB):
                sd((ngrp - 1) * _NB + b, b).wait()

        @pl.when(cid == 0)
        def _():
            run_side(j0, sid * j0)

        @pl.when(cid == 1)
        def _():
            run_side(j1, _NS * j0 + sid * j1)

        plsc.subcore_barrier()
        pltpu.sync_copy(acc_sh.at[pl.ds(sid * rpt, rpt)],
                        out_hbm.at[cid, pl.ds(sid * rpt, rpt)])

    return agg_kernel(h, src2d, dst2d, zeros)


def _tc_layer(hs, parts_list, w1, batch3, s1, a1, w2, s2, a2, bn, g):
    k = len(hs)
    n, hdim = hs[0].shape
    grid = (n // bn,)

    def body(*refs):
        h_refs = refs[:k]
        p_refs = refs[k:2 * k]
        w1_ref = refs[2 * k]
        b_ref, s1_ref, a1_ref, w2_ref, s2_ref, a2_ref, u_ref, pool_ref = \
            refs[2 * k + 1:]
        ms = [h_ref[...] + p_ref[0] + p_ref[1]
              for h_ref, p_ref in zip(h_refs, p_refs)]
        m = ms[0] if k == 1 else jnp.concatenate(ms, axis=1)
        acc = jnp.dot(m, w1_ref[...], preferred_element_type=jnp.float32)
        t = jnp.maximum(acc * s1_ref[...] + a1_ref[...], 0.0)
        u = jnp.dot(t, w2_ref[...], preferred_element_type=jnp.float32)
        u = jnp.maximum(u * s2_ref[...] + a2_ref[...], 0.0)
        u_ref[...] = u
        ids = b_ref[0, 0, :]
        oh = (ids[:, None] == lax.broadcasted_iota(jnp.int32, (1, g), 1))
        contrib = lax.dot_general(oh.astype(jnp.float32), u,
                                  (((0,), (0,)), ((), ())),
                                  preferred_element_type=jnp.float32)

        @pl.when(pl.program_id(0) == 0)
        def _():
            pool_ref[...] = jnp.zeros_like(pool_ref)

        pool_ref[...] += contrib

    vec = lambda: pl.BlockSpec((1, hdim), lambda i: (0, 0))
    in_specs = (
        [pl.BlockSpec((bn, hdim), lambda i: (i, 0))] * k
        + [pl.BlockSpec((2, bn, hdim), lambda i: (0, i, 0))] * k
        + [pl.BlockSpec((k * hdim, hdim), lambda i: (0, 0))]
        + [pl.BlockSpec((1, 1, bn), lambda i: (i, 0, 0)),
           vec(), vec(),
           pl.BlockSpec((hdim, hdim), lambda i: (0, 0)),
           vec(), vec()]
    )
    return pl.pallas_call(
        body,
        grid=grid,
        in_specs=in_specs,
        out_specs=[
            pl.BlockSpec((bn, hdim), lambda i: (i, 0)),
            pl.BlockSpec((g, hdim), lambda i: (0, 0)),
        ],
        out_shape=[
            jax.ShapeDtypeStruct((n, hdim), jnp.float32),
            jax.ShapeDtypeStruct((g, hdim), jnp.float32),
        ],
    )(*hs, *parts_list, w1, batch3, s1, a1, w2, s2, a2)


def _tc_classifier(p1, p2, p3, w1a, w1b, w1c, b1, w2, b2):
    g, hdim = p1.shape
    c = w2.shape[1]

    def body(p1r, p2r, p3r, w1ar, w1br, w1cr, b1r, w2r, b2r, out_ref):
        t = (jnp.dot(p1r[...], w1ar[...], preferred_element_type=jnp.float32)
             + jnp.dot(p2r[...], w1br[...], preferred_element_type=jnp.float32)
             + jnp.dot(p3r[...], w1cr[...], preferred_element_type=jnp.float32)
             + b1r[...])
        t = jnp.maximum(t, 0.0)
        out_ref[...] = (jnp.dot(t, w2r[...], preferred_element_type=jnp.float32)
                        + b2r[...])

    return pl.pallas_call(
        body,
        out_shape=jax.ShapeDtypeStruct((g, c), jnp.float32),
    )(p1, p2, p3, w1a, w1b, w1c, b1, w2, b2)


def kernel(x, edge_index, batch, mlp_W1, mlp_b1, mlp_g1, mlp_be1, mlp_W2,
           mlp_b2, bn_g, bn_b, cls_W1, cls_b1, cls_W2, cls_b2):
    n, f_in = x.shape
    e = edge_index.shape[1]
    g = 64
    layers = len(mlp_W1)
    hdim = mlp_W1[0].shape[1]

    j_per_tile = -(-(-(-e // (_NW * _CH))) // 8) * 8
    e_pad = _NW * j_per_tile * _CH
    j0 = 2 * j_per_tile - 8
    j1 = 2 * j_per_tile - j0
    pad = e_pad - e
    src2d = jnp.concatenate(
        [edge_index[0], jnp.zeros((pad,), jnp.int32)]).reshape(-1, _CH)
    dst2d = jnp.concatenate(
        [edge_index[1], jnp.full((pad,), n, jnp.int32)]).reshape(-1, _CH)
    n_pad = -(-(n + 1) // 128) * 128
    zeros = jnp.zeros((n_pad, hdim), jnp.float32)

    bn = 2000
    batch3 = batch.reshape(n // bn, 1, bn)

    sc = 1.0 / jnp.sqrt(jnp.float32(1.0 + BN_EPS))
    h = x
    pooled = []
    for i in range(layers):
        f = h.shape[1]
        if f == hdim:
            hs = [h]
        else:
            hs = [h[:, j * hdim:(j + 1) * hdim] for j in range(f // hdim)]
        parts_list = [_sc_agg(hh, src2d, dst2d, zeros, n_pad, j0, j1)
                      for hh in hs]
        s1 = (mlp_g1[i] * sc).reshape(1, -1)
        a1 = (mlp_b1[i] * mlp_g1[i] * sc + mlp_be1[i]).reshape(1, -1)
        s2 = (bn_g[i] * sc).reshape(1, -1)
        a2 = (mlp_b2[i] * bn_g[i] * sc + bn_b[i]).reshape(1, -1)
        h, pool = _tc_layer(hs, parts_list, mlp_W1[i], batch3, s1, a1,
                            mlp_W2[i], s2, a2, bn, g)
        pooled.append(pool)

    w1a, w1b, w1c = (cls_W1[:hdim], cls_W1[hdim:2 * hdim], cls_W1[2 * hdim:])
    return _tc_classifier(pooled[0], pooled[1], pooled[2], w1a, w1b, w1c,
                          cls_b1.reshape(1, -1), cls_W2, cls_b2.reshape(1, -1))
